# baseline (speedup 1.0000x reference)
import functools

import jax
import jax.numpy as jnp
from jax.experimental import pallas as pl
from jax.experimental.pallas import tpu as pltpu
from jax.experimental.pallas import tpu_sc as plsc

_LANES = 16


@jax.jit
def kernel(x, token_table, pos_table):
    B, L = x.shape
    V, D = token_table.shape
    N = B * L
    W = 8 * L

    x_flat = x.reshape(N).astype(jnp.int32)

    tok_t = token_table.T
    CW = 512
    V_main = (V // CW) * CW

    @functools.partial(
        pl.kernel,
        out_type=jax.ShapeDtypeStruct((V // 4, 128), jnp.float32),
        mesh=plsc.VectorSubcoreMesh(
            core_axis_name="core", subcore_axis_name="subcore"
        ),
        compiler_params=pltpu.CompilerParams(
            use_tc_tiling_on_sc=True, needs_layout_passes=False
        ),
    )
    def sc_pack(tokt_hbm, out_hbm):
        def body(in_vmem, o_vmem):
            lanes16 = jax.lax.iota(jnp.int32, _LANES)

            @pl.loop(0, CW)
            def _(v):
                col = jnp.full((_LANES,), v, jnp.int32)
                q = v // 4
                off = (v % 4) * D
                for c in range(0, D, _LANES):
                    vals = plsc.load_gather(in_vmem, [lanes16 + c, col])
                    o_vmem[q, pl.ds(off + c, _LANES)] = vals

        pltpu.emit_pipeline(
            body,
            grid=(V_main // CW,),
            in_specs=[pl.BlockSpec((D, CW), lambda i: (0, i))],
            out_specs=[pl.BlockSpec((CW // 4, 128), lambda i: (i, 0))],
            core_axis_name=("core", "subcore"),
            dimension_semantics=(pltpu.PARALLEL,),
        )(tokt_hbm, out_hbm)

    packed_main = sc_pack(tok_t)

    tail_vals = jax.lax.slice(token_table, (V_main, 0), (V, D))
    tok_lin = jax.lax.dynamic_update_slice(
        packed_main, tail_vals.reshape((V - V_main) // 4, 128), (V_main // 4, 0)
    ).reshape(V, D)

    @functools.partial(
        pl.kernel,
        out_type=jax.ShapeDtypeStruct((N, D), jnp.float32),
        mesh=plsc.VectorSubcoreMesh(
            core_axis_name="core", subcore_axis_name="subcore"
        ),
        compiler_params=pltpu.CompilerParams(use_tc_tiling_on_sc=False),
    )
    def sc_embed(tok_hbm, idx_hbm, out_hbm):
        def body(i_vmem, o_vmem):
            pltpu.sync_copy(tok_hbm.at[i_vmem], o_vmem)

        pltpu.emit_pipeline(
            body,
            grid=(N // W,),
            in_specs=[pl.BlockSpec((W,), lambda i: (i,))],
            out_specs=[pl.BlockSpec((W, D), lambda i: (i, 0))],
            core_axis_name=("core", "subcore"),
            dimension_semantics=(pltpu.PARALLEL,),
        )(idx_hbm, out_hbm)

    flat = sc_embed(tok_lin, x_flat)

    G = L // 4
    t2 = flat.reshape(N // 4, 128)
    BB = 256

    pos128 = pos_table.reshape(G, 128)

    def tc_body(in_ref, pos_ref, out_ref):
        v = in_ref[...].reshape(BB, G, 128)
        for g in range(G):
            out_ref[g] = v[:, g, :].T + pos_ref[g][:, None]

    out128 = pl.pallas_call(
        tc_body,
        grid=(B // BB,),
        in_specs=[
            pl.BlockSpec((G * BB, 128), lambda i: (i, 0)),
            pl.BlockSpec((G, 128), lambda i: (0, 0)),
        ],
        out_specs=pl.BlockSpec((G, 128, BB), lambda i: (0, 0, i)),
        out_shape=jax.ShapeDtypeStruct((G, 128, B), jnp.float32),
    )(t2, pos128)
    return out128.reshape(L, D, B).transpose(2, 0, 1)


# speedup vs baseline: 1.1802x; 1.1802x over previous
import functools

import jax
import jax.numpy as jnp
from jax.experimental import pallas as pl
from jax.experimental.pallas import tpu as pltpu
from jax.experimental.pallas import tpu_sc as plsc

_LANES = 16


@jax.jit
def kernel(x, token_table, pos_table):
    B, L = x.shape
    V, D = token_table.shape
    N = B * L
    W = 8 * L

    x_flat = x.reshape(N).astype(jnp.int32)

    tok_t = token_table.T
    CW = 512
    V_main = (V // CW) * CW

    @functools.partial(
        pl.kernel,
        out_type=jax.ShapeDtypeStruct((V // 4, 128), jnp.float32),
        mesh=plsc.VectorSubcoreMesh(
            core_axis_name="core", subcore_axis_name="subcore"
        ),
        compiler_params=pltpu.CompilerParams(
            use_tc_tiling_on_sc=True, needs_layout_passes=False
        ),
    )
    def sc_pack(tokt_hbm, out_hbm):
        def body(in_vmem, o_vmem):
            iota = jax.lax.iota(jnp.int32, _LANES)
            qbase = iota // 4
            jbase = (iota % 4) * D

            @pl.loop(0, CW, step=_LANES)
            def _(v0):
                rows = qbase + v0 // 4
                for d in range(D):
                    vals = in_vmem[d, pl.ds(v0, _LANES)]
                    plsc.store_scatter(o_vmem, [rows, jbase + d], vals)

        pltpu.emit_pipeline(
            body,
            grid=(V_main // CW,),
            in_specs=[pl.BlockSpec((D, CW), lambda i: (0, i))],
            out_specs=[pl.BlockSpec((CW // 4, 128), lambda i: (i, 0))],
            core_axis_name=("core", "subcore"),
            dimension_semantics=(pltpu.PARALLEL,),
        )(tokt_hbm, out_hbm)

    packed_main = sc_pack(tok_t)

    tail_vals = jax.lax.slice(token_table, (V_main, 0), (V, D))
    tok_lin = jax.lax.dynamic_update_slice(
        packed_main, tail_vals.reshape((V - V_main) // 4, 128), (V_main // 4, 0)
    ).reshape(V, D)

    @functools.partial(
        pl.kernel,
        out_type=jax.ShapeDtypeStruct((N, D), jnp.float32),
        mesh=plsc.VectorSubcoreMesh(
            core_axis_name="core", subcore_axis_name="subcore"
        ),
        compiler_params=pltpu.CompilerParams(use_tc_tiling_on_sc=False),
    )
    def sc_embed(tok_hbm, idx_hbm, out_hbm):
        def body(i_vmem, o_vmem):
            pltpu.sync_copy(tok_hbm.at[i_vmem], o_vmem)

        pltpu.emit_pipeline(
            body,
            grid=(N // W,),
            in_specs=[pl.BlockSpec((W,), lambda i: (i,))],
            out_specs=[pl.BlockSpec((W, D), lambda i: (i, 0))],
            core_axis_name=("core", "subcore"),
            dimension_semantics=(pltpu.PARALLEL,),
        )(idx_hbm, out_hbm)

    flat = sc_embed(tok_lin, x_flat)

    G = L // 4
    t2 = flat.reshape(N // 4, 128)
    BB = 256

    pos128 = pos_table.reshape(G, 128)

    def tc_body(in_ref, pos_ref, out_ref):
        v = in_ref[...].reshape(BB, G, 128)
        for g in range(G):
            out_ref[g] = v[:, g, :].T + pos_ref[g][:, None]

    out128 = pl.pallas_call(
        tc_body,
        grid=(B // BB,),
        in_specs=[
            pl.BlockSpec((G * BB, 128), lambda i: (i, 0)),
            pl.BlockSpec((G, 128), lambda i: (0, 0)),
        ],
        out_specs=pl.BlockSpec((G, 128, BB), lambda i: (0, 0, i)),
        out_shape=jax.ShapeDtypeStruct((G, 128, B), jnp.float32),
    )(t2, pos128)
    return out128.reshape(L, D, B).transpose(2, 0, 1)
